# Initial kernel scaffold; baseline (speedup 1.0000x reference)
import jax
import jax.numpy as jnp
from jax.experimental import pallas as pl


def kernel(inp_pos, out_pos, inp_features, neighbors_index, W1, b1, W2, b2, W3, b3):
    raise NotImplementedError("write your pallas kernel here")


# speedup vs baseline: 5.9153x; 5.9153x over previous
import functools

import jax
import jax.numpy as jnp
from jax import lax
from jax.experimental import pallas as pl
from jax.experimental.pallas import tpu as pltpu
from jax.experimental.pallas import tpu_sc as plsc

N_TILES = 32
LANES = 16
C1 = 32
C_CHUNK = 8


def _sc_segment_sum(tbx, tby, tbz, pos3, nbr3, wsp):
    t_rows = tbx.shape[0]
    k_nbr = nbr3.shape[1]
    npt = nbr3.shape[2]
    n_grp = npt // LANES
    mesh = plsc.VectorSubcoreMesh(core_axis_name="c", subcore_axis_name="s")

    @functools.partial(
        pl.kernel,
        out_type=jax.ShapeDtypeStruct((N_TILES, C1, npt), jnp.float32),
        mesh=mesh,
        compiler_params=pltpu.CompilerParams(needs_layout_passes=False),
        scratch_types=[
            pltpu.VMEM((t_rows,), jnp.float32),
            pltpu.VMEM((t_rows,), jnp.float32),
            pltpu.VMEM((t_rows,), jnp.float32),
            pltpu.VMEM((3, npt), jnp.float32),
            pltpu.VMEM((k_nbr, npt), jnp.int32),
            pltpu.VMEM((224, LANES), jnp.float32),
            pltpu.VMEM((C1, npt), jnp.float32),
        ],
    )
    def sc_kernel(tbx_hbm, tby_hbm, tbz_hbm, pos_hbm, nbr_hbm, wsp_hbm,
                  out_hbm, tbx_v, tby_v, tbz_v, pos_v, nbr_v, wsp_v, out_v):
        wid = lax.axis_index("s") * 2 + lax.axis_index("c")
        pltpu.sync_copy(tbx_hbm, tbx_v)
        pltpu.sync_copy(tby_hbm, tby_v)
        pltpu.sync_copy(tbz_hbm, tbz_v)
        pltpu.sync_copy(pos_hbm.at[wid], pos_v)
        pltpu.sync_copy(nbr_hbm.at[wid], nbr_v)
        pltpu.sync_copy(wsp_hbm, wsp_v)

        for chunk in range(C1 // C_CHUNK):
            ch0 = chunk * C_CHUNK
            w3 = [wsp_v[3 * C1 + ch0 + c, :] for c in range(C_CHUNK)]
            w4 = [wsp_v[4 * C1 + ch0 + c, :] for c in range(C_CHUNK)]
            w5 = [wsp_v[5 * C1 + ch0 + c, :] for c in range(C_CHUNK)]

            def g_body(g, carry, ch0=ch0, w3=w3, w4=w4, w5=w5):
                gs = pl.multiple_of(g * LANES, LANES)
                cx = pos_v[0, pl.ds(gs, LANES)]
                cy = pos_v[1, pl.ds(gs, LANES)]
                cz = pos_v[2, pl.ds(gs, LANES)]
                cc = [cx * wsp_v[0 * C1 + ch0 + c, :]
                      + cy * wsp_v[1 * C1 + ch0 + c, :]
                      + cz * wsp_v[2 * C1 + ch0 + c, :]
                      + wsp_v[6 * C1 + ch0 + c, :]
                      for c in range(C_CHUNK)]
                accs = [jnp.zeros((LANES,), jnp.float32)] * C_CHUNK
                for k in range(k_nbr):
                    idx = nbr_v[k, pl.ds(gs, LANES)]
                    nx = plsc.load_gather(tbx_v, [idx])
                    ny = plsc.load_gather(tby_v, [idx])
                    nz = plsc.load_gather(tbz_v, [idx])
                    for c in range(C_CHUNK):
                        t = nx * w3[c] + ny * w4[c] + nz * w5[c] + cc[c]
                        accs[c] = accs[c] + jnp.maximum(t, 0.0)
                for c in range(C_CHUNK):
                    out_v[ch0 + c, pl.ds(gs, LANES)] = accs[c]
                return carry

            lax.fori_loop(0, n_grp, g_body, 0)

        pltpu.sync_copy(out_v, out_hbm.at[wid])

    return sc_kernel(tbx, tby, tbz, pos3, nbr3, wsp)


def _tc_project(s3, W2, b2, W3, b3, k_nbr):
    def tc_body(s_ref, w2_ref, b2_ref, w3_ref, b3_ref, o_ref):
        hi = jax.lax.Precision.HIGHEST
        wc = jnp.dot(w2_ref[...], w3_ref[...], precision=hi,
                     preferred_element_type=jnp.float32)
        bias = (float(k_nbr) * jnp.dot(b2_ref[...][None, :], w3_ref[...],
                                       precision=hi,
                                       preferred_element_type=jnp.float32)
                + b3_ref[...][None, :])
        o_ref[...] = lax.dot_general(
            s_ref[...], wc, (((1,), (0,)), ((), ())),
            precision=hi, preferred_element_type=jnp.float32) + bias

    nt, _, npt = s3.shape
    return pl.pallas_call(
        tc_body,
        out_shape=jax.ShapeDtypeStruct((nt, npt, 32), jnp.float32),
    )(s3, W2, b2, W3, b3)


def kernel(inp_pos, out_pos, inp_features, neighbors_index,
           W1, b1, W2, b2, W3, b3):
    n = inp_pos.shape[0]
    k_nbr = neighbors_index.shape[1]
    npad = ((n + LANES * N_TILES - 1) // (LANES * N_TILES)) * (LANES * N_TILES)
    npt = npad // N_TILES

    t_rows = ((n + 1 + 7) // 8) * 8
    tbl = jnp.zeros((t_rows, 3), jnp.float32).at[1:n + 1].set(inp_pos)
    tbx, tby, tbz = tbl[:, 0], tbl[:, 1], tbl[:, 2]

    pos3 = (jnp.zeros((npad, 3), jnp.float32).at[:n].set(inp_pos)
            .reshape(N_TILES, npt, 3).transpose(0, 2, 1))
    nbr3 = (jnp.zeros((npad, k_nbr), jnp.int32).at[:n].set(neighbors_index)
            .reshape(N_TILES, npt, k_nbr).transpose(0, 2, 1))

    w_rows = jnp.concatenate([W1.astype(jnp.float32).reshape(-1),
                              b1.astype(jnp.float32)])
    wsp = jnp.broadcast_to(w_rows[:, None], (w_rows.shape[0], LANES))

    s3 = _sc_segment_sum(tbx, tby, tbz, pos3, nbr3, wsp)
    out = _tc_project(s3, W2, b2, W3, b3, k_nbr)
    return out.reshape(npad, 32)[:n].reshape(n, 1, 1, 32)


# speedup vs baseline: 6.4744x; 1.0945x over previous
import functools

import jax
import jax.numpy as jnp
from jax import lax
from jax.experimental import pallas as pl
from jax.experimental.pallas import tpu as pltpu
from jax.experimental.pallas import tpu_sc as plsc

N_TILES = 32
LANES = 16
C1 = 32
NQ = 4
NR = N_TILES // NQ
CQ = C1 // NQ


def _tc_pre(tbl_t, pos_t, W1, b1):
    def body(tbl_ref, pos_ref, w1_ref, b1_ref, p_ref, c_ref):
        w = w1_ref[...]
        tb = tbl_ref[...]
        ps = pos_ref[...]
        p_ref[...] = (w[3, :][:, None] * tb[0:1, :]
                      + w[4, :][:, None] * tb[1:2, :]
                      + w[5, :][:, None] * tb[2:3, :])
        c_ref[...] = (w[0, :][:, None] * ps[0:1, :]
                      + w[1, :][:, None] * ps[1:2, :]
                      + w[2, :][:, None] * ps[2:3, :]
                      + b1_ref[...][:, None])

    t_rows = tbl_t.shape[1]
    npad = pos_t.shape[1]
    return pl.pallas_call(
        body,
        out_shape=(jax.ShapeDtypeStruct((C1, t_rows), jnp.float32),
                   jax.ShapeDtypeStruct((C1, npad), jnp.float32)),
    )(tbl_t, pos_t, W1, b1)


def _sc_segment_sum(p2, c4, nbr5, t_rows):
    k_nbr = nbr5.shape[1]
    npr = nbr5.shape[2]
    n_grp = npr // LANES
    mesh = plsc.VectorSubcoreMesh(core_axis_name="c", subcore_axis_name="s")

    @functools.partial(
        pl.kernel,
        out_type=jax.ShapeDtypeStruct((NQ, NR, CQ, npr), jnp.float32),
        mesh=mesh,
        compiler_params=pltpu.CompilerParams(needs_layout_passes=False),
        scratch_types=[
            pltpu.VMEM((CQ * t_rows,), jnp.float32),
            pltpu.VMEM((CQ, npr), jnp.float32),
            pltpu.VMEM((k_nbr, npr), jnp.int32),
            pltpu.VMEM((CQ, npr), jnp.float32),
        ],
    )
    def sc_kernel(p_hbm, c_hbm, nbr_hbm, out_hbm, p_v, c_v, nbr_v, out_v):
        wid = lax.axis_index("s") * 2 + lax.axis_index("c")
        q = wid // NR
        r = wid - q * NR
        pltpu.sync_copy(p_hbm.at[q], p_v)
        pltpu.sync_copy(c_hbm.at[q, r], c_v)
        pltpu.sync_copy(nbr_hbm.at[r], nbr_v)

        def g_body(g, carry):
            gs = pl.multiple_of(g * LANES, LANES)
            ccs = [c_v[c, pl.ds(gs, LANES)] for c in range(CQ)]
            accs = [jnp.zeros((LANES,), jnp.float32)] * CQ
            for k in range(k_nbr):
                idx = nbr_v[k, pl.ds(gs, LANES)]
                for c in range(CQ):
                    pv = plsc.load_gather(p_v, [idx + (c * t_rows)])
                    accs[c] = accs[c] + jnp.maximum(pv + ccs[c], 0.0)
            for c in range(CQ):
                out_v[c, pl.ds(gs, LANES)] = accs[c]
            return carry

        lax.fori_loop(0, n_grp, g_body, 0)
        pltpu.sync_copy(out_v, out_hbm.at[q, r])

    return sc_kernel(p2, c4, nbr5)


def _tc_project(s_nm, W2, b2, W3, b3, k_nbr):
    def body(s_ref, w2_ref, b2_ref, w3_ref, b3_ref, o_ref):
        hi = jax.lax.Precision.HIGHEST
        wc = jnp.dot(w2_ref[...], w3_ref[...], precision=hi,
                     preferred_element_type=jnp.float32)
        bias = (float(k_nbr) * jnp.dot(b2_ref[...][None, :], w3_ref[...],
                                       precision=hi,
                                       preferred_element_type=jnp.float32)
                + b3_ref[...][None, :])
        o_ref[...] = jnp.dot(s_ref[...], wc,
                             preferred_element_type=jnp.float32) + bias

    npad = s_nm.shape[0]
    return pl.pallas_call(
        body,
        out_shape=jax.ShapeDtypeStruct((npad, 32), jnp.float32),
    )(s_nm, W2, b2, W3, b3)


def kernel(inp_pos, out_pos, inp_features, neighbors_index,
           W1, b1, W2, b2, W3, b3):
    n = inp_pos.shape[0]
    k_nbr = neighbors_index.shape[1]
    npad = ((n + LANES * N_TILES - 1) // (LANES * N_TILES)) * (LANES * N_TILES)
    npr = npad // NR

    t_rows = n + 1
    tbl_t = jnp.zeros((3, t_rows), jnp.float32).at[:, 1:].set(inp_pos.T)
    pos_t = jnp.zeros((3, npad), jnp.float32).at[:, :n].set(inp_pos.T)

    p_tab, c_tab = _tc_pre(tbl_t, pos_t, W1.astype(jnp.float32),
                           b1.astype(jnp.float32))

    p2 = p_tab.reshape(NQ, CQ * t_rows)
    c4 = c_tab.reshape(NQ, CQ, NR, npr).transpose(0, 2, 1, 3)
    nbr5 = (jnp.zeros((npad, k_nbr), jnp.int32).at[:n].set(neighbors_index)
            .reshape(NR, npr, k_nbr).transpose(0, 2, 1))

    out5 = _sc_segment_sum(p2, c4, nbr5, t_rows)

    s_nm = out5.transpose(1, 3, 0, 2).reshape(npad, C1)
    out = _tc_project(s_nm, W2, b2, W3, b3, k_nbr)
    return out[:n].reshape(n, 1, 1, 32)


# speedup vs baseline: 6.6267x; 1.0235x over previous
import functools

import jax
import jax.numpy as jnp
from jax import lax
from jax.experimental import pallas as pl
from jax.experimental.pallas import tpu as pltpu
from jax.experimental.pallas import tpu_sc as plsc

N_TILES = 32
LANES = 16
C1 = 32
NQ = 4
NR = N_TILES // NQ
CQ = C1 // NQ


def _tc_pre(pos_t, W1, b1, W2, b2, W3, b3, k_nbr):
    def body(pos_ref, w1_ref, b1_ref, w2_ref, b2_ref, w3_ref, b3_ref,
             p_ref, c_ref, wc_ref, bias_ref):
        w = w1_ref[...]
        ps = pos_ref[...]
        p_ref[...] = (w[3, :][:, None] * ps[0:1, :]
                      + w[4, :][:, None] * ps[1:2, :]
                      + w[5, :][:, None] * ps[2:3, :])
        c_ref[...] = (w[0, :][:, None] * ps[0:1, :]
                      + w[1, :][:, None] * ps[1:2, :]
                      + w[2, :][:, None] * ps[2:3, :]
                      + b1_ref[...][:, None])
        hi = jax.lax.Precision.HIGHEST
        wc_ref[...] = jnp.dot(w2_ref[...], w3_ref[...], precision=hi,
                              preferred_element_type=jnp.float32)
        bias_ref[...] = (float(k_nbr)
                         * jnp.dot(b2_ref[...][None, :], w3_ref[...],
                                   precision=hi,
                                   preferred_element_type=jnp.float32)
                         + b3_ref[...][None, :])

    npad = pos_t.shape[1]
    return pl.pallas_call(
        body,
        out_shape=(jax.ShapeDtypeStruct((C1, npad), jnp.float32),
                   jax.ShapeDtypeStruct((C1, npad), jnp.float32),
                   jax.ShapeDtypeStruct((C1, C1), jnp.float32),
                   jax.ShapeDtypeStruct((1, C1), jnp.float32)),
    )(pos_t, W1, b1, W2, b2, W3, b3)


def _sc_segment_sum(p3, c3, nbr_t):
    k_nbr = nbr_t.shape[0]
    npad = nbr_t.shape[1]
    npr = npad // NR
    n_grp = npr // LANES
    mesh = plsc.VectorSubcoreMesh(core_axis_name="c", subcore_axis_name="s")

    @functools.partial(
        pl.kernel,
        out_type=jax.ShapeDtypeStruct((NQ, CQ, npad), jnp.float32),
        mesh=mesh,
        compiler_params=pltpu.CompilerParams(needs_layout_passes=False),
        scratch_types=[
            pltpu.VMEM((CQ, npad), jnp.float32),
            pltpu.VMEM((CQ, npr), jnp.float32),
            pltpu.VMEM((k_nbr, npr), jnp.int32),
            pltpu.VMEM((CQ, npr), jnp.float32),
            pltpu.SemaphoreType.DMA,
        ],
    )
    def sc_kernel(p_hbm, c_hbm, nbr_hbm, out_hbm, p_v, c_v, nbr_v, out_v,
                  sem):
        wid = lax.axis_index("s") * 2 + lax.axis_index("c")
        q = wid // NR
        r = wid - q * NR
        base = r * npr

        cps = [pltpu.async_copy(p_hbm.at[q], p_v, sem),
               pltpu.async_copy(c_hbm.at[q, :, pl.ds(base, npr)], c_v, sem),
               pltpu.async_copy(nbr_hbm.at[:, pl.ds(base, npr)], nbr_v, sem)]
        for cp in cps:
            cp.wait()

        def g_body(g, carry):
            gs = pl.multiple_of(g * LANES, LANES)
            ccs = [c_v[c, pl.ds(gs, LANES)] for c in range(CQ)]
            accs = [jnp.zeros((LANES,), jnp.float32)] * CQ
            zero = jnp.zeros((LANES,), jnp.float32)
            for k in range(k_nbr):
                idx = nbr_v[k, pl.ds(gs, LANES)]
                valid = idx >= 1
                jm1 = jnp.maximum(idx - 1, 0)
                for c in range(CQ):
                    pv = plsc.load_gather(
                        p_v, [jnp.full((LANES,), c, jnp.int32), jm1])
                    pv = jnp.where(valid, pv, zero)
                    accs[c] = accs[c] + jnp.maximum(pv + ccs[c], 0.0)
            for c in range(CQ):
                out_v[c, pl.ds(gs, LANES)] = accs[c]
            return carry

        lax.fori_loop(0, n_grp, g_body, 0)
        pltpu.sync_copy(out_v, out_hbm.at[q, :, pl.ds(base, npr)])

    return sc_kernel(p3, c3, nbr_t)


def _tc_project(s_cm, wc, bias):
    def body(s_ref, wc_ref, bias_ref, o_ref):
        o_ref[...] = lax.dot_general(
            s_ref[...], wc_ref[...], (((0,), (0,)), ((), ())),
            preferred_element_type=jnp.float32) + bias_ref[...]

    npad = s_cm.shape[1]
    return pl.pallas_call(
        body,
        out_shape=jax.ShapeDtypeStruct((npad, C1), jnp.float32),
    )(s_cm, wc, bias)


def kernel(inp_pos, out_pos, inp_features, neighbors_index,
           W1, b1, W2, b2, W3, b3):
    n = inp_pos.shape[0]
    k_nbr = neighbors_index.shape[1]
    npad = ((n + LANES * N_TILES - 1) // (LANES * N_TILES)) * (LANES * N_TILES)

    pos_t = jnp.zeros((3, npad), jnp.float32).at[:, :n].set(
        inp_pos.astype(jnp.float32).T)
    nbr_t = jnp.zeros((k_nbr, npad), jnp.int32).at[:, :n].set(
        neighbors_index.T)

    p_tab, c_tab, wc, bias = _tc_pre(pos_t, W1.astype(jnp.float32),
                                     b1.astype(jnp.float32), W2, b2, W3, b3,
                                     k_nbr)
    p3 = p_tab.reshape(NQ, CQ, npad)
    c3 = c_tab.reshape(NQ, CQ, npad)
    s3 = _sc_segment_sum(p3, c3, nbr_t)
    out = _tc_project(s3.reshape(C1, npad), wc, bias)
    return out[:n].reshape(n, 1, 1, C1)
